# baseline (speedup 1.0000x reference)
import jax
import jax.numpy as jnp
from jax import lax
from jax.experimental import pallas as pl
from jax.experimental.pallas import tpu as pltpu, tpu_sc as plsc

BATCH = 16384
MAX_LEN = 50
EMBED_DIM = 32
ITEMS = 1000001
NC, NS, L = 2, 16, 16
NW = NC * NS

BLK = 768
NFULL = ITEMS // BLK
REM_START = NFULL * BLK
REM = ITEMS - REM_START
ITEMS_PAD = 1000032
PITCH = 32
PACKED = ITEMS_PAD * PITCH

ROWS_W = BATCH // NW
CHUNK = 16
NCHUNK = ROWS_W // CHUNK
IDX_W = ROWS_W * MAX_LEN
IDX_C = CHUNK * MAX_LEN


def _relayout_kernel(tab_t_hbm, rem_hbm, packed_hbm,
                     in_a, in_b, out_a, out_b,
                     isem_a, isem_b, osem_a, osem_b):
    wid = lax.axis_index("s") * NC + lax.axis_index("c")
    iota = lax.iota(jnp.int32, L)
    masks = {b: lax.ne(lax.bitwise_and(iota, b), 0) for b in (1, 2, 4, 8)}
    pminus = {b: lax.bitwise_and(lax.sub(iota, b), L - 1) for b in (1, 2, 4, 8)}
    pplus = {b: lax.bitwise_and(lax.add(iota, b), L - 1) for b in (1, 2, 4, 8)}

    def fire(blk, in_v, isem):
        for tr in range(4):
            pltpu.async_copy(
                tab_t_hbm.at[pl.ds(8 * tr, 8), pl.ds(blk * BLK, BLK)],
                in_v.at[tr], isem)

    def wait_in(blk, in_v, isem):
        for tr in range(4):
            pltpu.make_async_copy(
                tab_t_hbm.at[pl.ds(8 * tr, 8), pl.ds(blk * BLK, BLK)],
                in_v.at[tr], isem).wait()

    def transpose(blk, in_v, out_v, osem):
        def grp_body(g, _):
            i0 = g * L
            for half in range(2):
                d0 = half * L
                v = [in_v[(d0 + r) // 8, (d0 + r) % 8, pl.ds(i0, L)]
                     for r in range(L)]
                for b in (1, 2, 4, 8):
                    m, pm, pp = masks[b], pminus[b], pplus[b]
                    for r in range(L):
                        if r & b:
                            continue
                        a, c = v[r], v[r + b]
                        c_sh = c.at[pm].get(mode="promise_in_bounds")
                        a_sh = a.at[pp].get(mode="promise_in_bounds")
                        v[r] = jnp.where(m, c_sh, a)
                        v[r + b] = jnp.where(m, c, a_sh)
                for r in range(L):
                    out_v[pl.ds((i0 + r) * PITCH + d0, L)] = v[r]
            return 0
        lax.fori_loop(0, BLK // L, grp_body, 0)
        pltpu.async_copy(
            out_v, packed_hbm.at[pl.ds(blk * BLK * PITCH, BLK * PITCH)],
            osem)

    def wait_out(blk, out_v, osem):
        pltpu.make_async_copy(
            out_v, packed_hbm.at[pl.ds(blk * BLK * PITCH, BLK * PITCH)],
            osem).wait()

    b0 = wid
    fire(b0, in_a, isem_a)

    def pair_body(t, _):
        ba = b0 + 2 * t * NW
        bb = ba + NW

        @pl.when(bb < NFULL)
        def _():
            fire(bb, in_b, isem_b)

        @pl.when(ba < NFULL)
        def _():
            wait_in(ba, in_a, isem_a)

            @pl.when(t > 0)
            def _():
                wait_out(ba - 2 * NW, out_a, osem_a)

            transpose(ba, in_a, out_a, osem_a)

        @pl.when(ba + 2 * NW < NFULL)
        def _():
            fire(ba + 2 * NW, in_a, isem_a)

        @pl.when(bb < NFULL)
        def _():
            wait_in(bb, in_b, isem_b)

            @pl.when(t > 0)
            def _():
                wait_out(bb - 2 * NW, out_b, osem_b)

            transpose(bb, in_b, out_b, osem_b)

        return 0

    npair = (NFULL // NW + 2) // 2
    lax.fori_loop(0, npair, pair_body, 0)

    k = (NFULL - 1 - b0) // NW
    for parity, out_v, osem in ((0, out_a, osem_a), (1, out_b, osem_b)):
        k_par = k - lax.rem(k - parity, 2)

        @pl.when(k_par >= 0)
        def _():
            wait_out(b0 + k_par * NW, out_v, osem)

    @pl.when(wid == 5)
    def _():
        pltpu.sync_copy(rem_hbm, out_a.at[pl.ds(0, REM * PITCH)])
        pltpu.sync_copy(out_a.at[pl.ds(0, REM * PITCH)],
                        packed_hbm.at[pl.ds(REM_START * PITCH, REM * PITCH)])


def _sbpr_kernel(seq_hbm, next_hbm, table_hbm, out_hbm,
                 idx_v, next_idx_v, rows_a, rows_b, next_a, next_b,
                 prod_v, out_v, sem_a, sem_b, nsem_a, nsem_b):
    wid = lax.axis_index("s") * NC + lax.axis_index("c")

    pltpu.sync_copy(seq_hbm.at[pl.ds(wid * IDX_W, IDX_W)], idx_v)
    pltpu.sync_copy(next_hbm.at[pl.ds(wid * ROWS_W, ROWS_W)], next_idx_v)

    iota = lax.iota(jnp.int32, L)
    lane50 = lax.mul(iota, MAX_LEN)
    d16 = lax.mul(iota, L)

    def fire(c, rows_v, next_rows_v, sem, nsem):
        pltpu.async_copy(table_hbm.at[idx_v.at[pl.ds(c * IDX_C, IDX_C)]],
                         rows_v, sem)
        pltpu.async_copy(table_hbm.at[next_idx_v.at[pl.ds(c * CHUNK, CHUNK)]],
                         next_rows_v, nsem)

    def wait(c, rows_v, next_rows_v, sem, nsem):
        pltpu.make_async_copy(table_hbm.at[idx_v.at[pl.ds(c * IDX_C, IDX_C)]],
                              rows_v, sem).wait()
        pltpu.make_async_copy(
            table_hbm.at[next_idx_v.at[pl.ds(c * CHUNK, CHUNK)]],
            next_rows_v, nsem).wait()

    def compute(c, rows_v, next_rows_v):
        pos0 = lax.add(lane50, c * IDX_C)
        cnt = jnp.zeros((L,), jnp.float32)
        one = jnp.ones((L,), jnp.float32)
        zero = jnp.zeros((L,), jnp.float32)
        for j in range(MAX_LEN):
            v = plsc.load_gather(idx_v, [lax.add(pos0, j)])
            cnt = lax.add(cnt, lax.select(lax.ne(v, 0), one, zero))

        def row_body(r, _):
            b = r * MAX_LEN
            a0 = rows_v[b, pl.ds(0, L)]
            a1 = rows_v[b, pl.ds(L, L)]
            for j in range(1, MAX_LEN):
                a0 = lax.add(a0, rows_v[b + j, pl.ds(0, L)])
                a1 = lax.add(a1, rows_v[b + j, pl.ds(L, L)])
            p0 = lax.mul(a0, next_rows_v[r, pl.ds(0, L)])
            p1 = lax.mul(a1, next_rows_v[r, pl.ds(L, L)])
            plsc.store_scatter(prod_v, [lax.add(d16, r)], p0)
            plsc.store_scatter(prod_v, [lax.add(d16, r + L * L)], p1)
            return 0

        lax.fori_loop(0, CHUNK, row_body, 0)

        score = prod_v[pl.ds(0, L)]
        for d in range(1, EMBED_DIM):
            score = lax.add(score, prod_v[pl.ds(d * L, L)])
        out_v[pl.ds(c * CHUNK, CHUNK)] = lax.div(score, cnt)

    fire(0, rows_a, next_a, sem_a, nsem_a)

    def pair_body(t, _):
        c0 = t * 2
        fire(c0 + 1, rows_b, next_b, sem_b, nsem_b)
        wait(c0, rows_a, next_a, sem_a, nsem_a)
        compute(c0, rows_a, next_a)

        @pl.when(t < NCHUNK // 2 - 1)
        def _():
            fire(c0 + 2, rows_a, next_a, sem_a, nsem_a)

        wait(c0 + 1, rows_b, next_b, sem_b, nsem_b)
        compute(c0 + 1, rows_b, next_b)
        return 0

    lax.fori_loop(0, NCHUNK // 2, pair_body, 0)
    pltpu.sync_copy(out_v, out_hbm.at[pl.ds(wid * ROWS_W, ROWS_W)])


def kernel(item_seq, next_item, item_embed):
    mesh = plsc.VectorSubcoreMesh(core_axis_name="c", subcore_axis_name="s",
                                  num_cores=NC, num_subcores=NS)

    relayout = pl.kernel(
        _relayout_kernel,
        out_type=jax.ShapeDtypeStruct((PACKED,), jnp.float32),
        mesh=mesh,
        compiler_params=pltpu.CompilerParams(needs_layout_passes=False,
                                             use_tc_tiling_on_sc=True),
        scratch_types=[
            pltpu.VMEM((4, 8, BLK), jnp.float32),
            pltpu.VMEM((4, 8, BLK), jnp.float32),
            pltpu.VMEM((BLK * PITCH,), jnp.float32),
            pltpu.VMEM((BLK * PITCH,), jnp.float32),
            pltpu.SemaphoreType.DMA,
            pltpu.SemaphoreType.DMA,
            pltpu.SemaphoreType.DMA,
            pltpu.SemaphoreType.DMA,
        ],
    )
    leftover = jnp.pad(item_embed[REM_START:],
                       ((0, 0), (0, PITCH - EMBED_DIM))).reshape(REM * PITCH)
    packed = relayout(jnp.transpose(item_embed), leftover)

    score = pl.kernel(
        _sbpr_kernel,
        out_type=jax.ShapeDtypeStruct((BATCH,), jnp.float32),
        mesh=mesh,
        compiler_params=pltpu.CompilerParams(needs_layout_passes=False,
                                             use_tc_tiling_on_sc=False),
        scratch_types=[
            pltpu.VMEM((IDX_W,), jnp.int32),
            pltpu.VMEM((ROWS_W,), jnp.int32),
            pltpu.VMEM((IDX_C, PITCH), jnp.float32),
            pltpu.VMEM((IDX_C, PITCH), jnp.float32),
            pltpu.VMEM((CHUNK, PITCH), jnp.float32),
            pltpu.VMEM((CHUNK, PITCH), jnp.float32),
            pltpu.VMEM((CHUNK * EMBED_DIM,), jnp.float32),
            pltpu.VMEM((ROWS_W,), jnp.float32),
            pltpu.SemaphoreType.DMA,
            pltpu.SemaphoreType.DMA,
            pltpu.SemaphoreType.DMA,
            pltpu.SemaphoreType.DMA,
        ],
    )
    seq_flat = item_seq.reshape(BATCH * MAX_LEN)
    table_rows = packed.reshape(ITEMS_PAD, PITCH)
    return score(seq_flat, next_item, table_rows)


# speedup vs baseline: 1.1279x; 1.1279x over previous
import jax
import jax.numpy as jnp
from jax import lax
from jax.experimental import pallas as pl
from jax.experimental.pallas import tpu as pltpu, tpu_sc as plsc

BATCH = 16384
MAX_LEN = 50
EMBED_DIM = 32
ITEMS = 1000001
NC, NS, L = 2, 16, 16
NW = NC * NS

BLK = 512
NFULL = ITEMS // BLK
REM_START = NFULL * BLK
REM = ITEMS - REM_START
ITEMS_PAD = 1000032
REM_PAD = ITEMS_PAD - REM_START
PITCH = 32
WROW = EMBED_DIM // 2
PACKED = ITEMS_PAD * WROW

ROWS_W = BATCH // NW
CHUNK = 16
NCHUNK = ROWS_W // CHUNK
IDX_W = ROWS_W * MAX_LEN
IDX_C = CHUNK * MAX_LEN


def _relayout_kernel(tab_t_hbm, rem_hbm, packed_hbm,
                     in_a, in_b, out_a, out_b,
                     isem_a, isem_b, osem_a, osem_b):
    wid = lax.axis_index("s") * NC + lax.axis_index("c")
    iota = lax.iota(jnp.int32, L)
    masks = {b: lax.ne(lax.bitwise_and(iota, b), 0) for b in (1, 2, 4, 8)}
    pminus = {b: lax.bitwise_and(lax.sub(iota, b), L - 1) for b in (1, 2, 4, 8)}
    pplus = {b: lax.bitwise_and(lax.add(iota, b), L - 1) for b in (1, 2, 4, 8)}

    def fire(blk, in_v, isem):
        for tr in range(4):
            pltpu.async_copy(
                tab_t_hbm.at[pl.ds(8 * tr, 8), pl.ds(blk * BLK, BLK)],
                in_v.at[tr], isem)

    def wait_in(blk, in_v, isem):
        for tr in range(4):
            pltpu.make_async_copy(
                tab_t_hbm.at[pl.ds(8 * tr, 8), pl.ds(blk * BLK, BLK)],
                in_v.at[tr], isem).wait()

    def transpose(blk, in_v, out_v, osem):
        def butterfly(v):
            for b in (1, 2, 4, 8):
                m, pm, pp = masks[b], pminus[b], pplus[b]
                for r in range(L):
                    if r & b:
                        continue
                    a, c = v[r], v[r + b]
                    c_sh = c.at[pm].get(mode="promise_in_bounds")
                    a_sh = a.at[pp].get(mode="promise_in_bounds")
                    v[r] = jnp.where(m, c_sh, a)
                    v[r + b] = jnp.where(m, c, a_sh)

        def grp_body(g, _):
            i0 = g * L
            v0 = [in_v[r // 8, r % 8, pl.ds(i0, L)] for r in range(L)]
            v1 = [in_v[2 + r // 8, r % 8, pl.ds(i0, L)] for r in range(L)]
            butterfly(v0)
            butterfly(v1)
            for r in range(L):
                pk = plsc.bitcast(
                    plsc.pack(v0[r], v1[r],
                              format=plsc.PackFormat.INTERLEAVED),
                    jnp.int32)
                out_v[pl.ds((i0 + r) * WROW, L)] = pk
            return 0
        lax.fori_loop(0, BLK // L, grp_body, 0)
        pltpu.async_copy(
            out_v, packed_hbm.at[pl.ds(blk * BLK * WROW, BLK * WROW)],
            osem)

    def wait_out(blk, out_v, osem):
        pltpu.make_async_copy(
            out_v, packed_hbm.at[pl.ds(blk * BLK * WROW, BLK * WROW)],
            osem).wait()

    b0 = wid
    fire(b0, in_a, isem_a)

    def pair_body(t, _):
        ba = b0 + 2 * t * NW
        bb = ba + NW

        @pl.when(bb < NFULL)
        def _():
            fire(bb, in_b, isem_b)

        @pl.when(ba < NFULL)
        def _():
            wait_in(ba, in_a, isem_a)

            @pl.when(t > 0)
            def _():
                wait_out(ba - 2 * NW, out_a, osem_a)

            transpose(ba, in_a, out_a, osem_a)

        @pl.when(ba + 2 * NW < NFULL)
        def _():
            fire(ba + 2 * NW, in_a, isem_a)

        @pl.when(bb < NFULL)
        def _():
            wait_in(bb, in_b, isem_b)

            @pl.when(t > 0)
            def _():
                wait_out(bb - 2 * NW, out_b, osem_b)

            transpose(bb, in_b, out_b, osem_b)

        return 0

    npair = (NFULL // NW + 2) // 2
    lax.fori_loop(0, npair, pair_body, 0)

    k = (NFULL - 1 - b0) // NW
    for parity, out_v, osem in ((0, out_a, osem_a), (1, out_b, osem_b)):
        k_par = k - lax.rem(k - parity, 2)

        @pl.when(k_par >= 0)
        def _():
            wait_out(b0 + k_par * NW, out_v, osem)

    @pl.when(wid == 5)
    def _():
        pltpu.sync_copy(rem_hbm, out_a.at[pl.ds(0, REM_PAD * WROW)])
        pltpu.sync_copy(out_a.at[pl.ds(0, REM_PAD * WROW)],
                        packed_hbm.at[pl.ds(REM_START * WROW,
                                            REM_PAD * WROW)])


def _sbpr_kernel(seq_hbm, next_hbm, table_hbm, out_hbm,
                 idx_v, next_idx_v, rows_a, rows_b, next_a, next_b,
                 prod_v, out_v, sem_a, sem_b, nsem_a, nsem_b):
    wid = lax.axis_index("s") * NC + lax.axis_index("c")

    pltpu.sync_copy(seq_hbm.at[pl.ds(wid * IDX_W, IDX_W)], idx_v)
    pltpu.sync_copy(next_hbm.at[pl.ds(wid * ROWS_W, ROWS_W)], next_idx_v)

    iota = lax.iota(jnp.int32, L)
    lane50 = lax.mul(iota, MAX_LEN)
    d16 = lax.mul(iota, L)

    def fire(c, rows_v, next_rows_v, sem, nsem):
        pltpu.async_copy(table_hbm.at[idx_v.at[pl.ds(c * IDX_C, IDX_C)]],
                         rows_v, sem)
        pltpu.async_copy(table_hbm.at[next_idx_v.at[pl.ds(c * CHUNK, CHUNK)]],
                         next_rows_v, nsem)

    def wait(c, rows_v, next_rows_v, sem, nsem):
        pltpu.make_async_copy(table_hbm.at[idx_v.at[pl.ds(c * IDX_C, IDX_C)]],
                              rows_v, sem).wait()
        pltpu.make_async_copy(
            table_hbm.at[next_idx_v.at[pl.ds(c * CHUNK, CHUNK)]],
            next_rows_v, nsem).wait()

    def compute(c, rows_v, next_rows_v):
        pos0 = lax.add(lane50, c * IDX_C)
        cnt = jnp.zeros((L,), jnp.float32)
        one = jnp.ones((L,), jnp.float32)
        zero = jnp.zeros((L,), jnp.float32)
        for j in range(MAX_LEN):
            v = plsc.load_gather(idx_v, [lax.add(pos0, j)])
            cnt = lax.add(cnt, lax.select(lax.ne(v, 0), one, zero))

        def row_body(r, _):
            b = r * MAX_LEN
            def row_dims(ref, q):
                w = plsc.bitcast(ref[q], jnp.bfloat16)
                return plsc.unpack(w, format=plsc.PackFormat.INTERLEAVED)

            a0, a1 = row_dims(rows_v, b)
            for j in range(1, MAX_LEN):
                e0, e1 = row_dims(rows_v, b + j)
                a0 = lax.add(a0, e0)
                a1 = lax.add(a1, e1)
            n0, n1 = row_dims(next_rows_v, r)
            p0 = lax.mul(a0, n0)
            p1 = lax.mul(a1, n1)
            plsc.store_scatter(prod_v, [lax.add(d16, r)], p0)
            plsc.store_scatter(prod_v, [lax.add(d16, r + L * L)], p1)
            return 0

        lax.fori_loop(0, CHUNK, row_body, 0)

        score = prod_v[pl.ds(0, L)]
        for d in range(1, EMBED_DIM):
            score = lax.add(score, prod_v[pl.ds(d * L, L)])
        out_v[pl.ds(c * CHUNK, CHUNK)] = lax.div(score, cnt)

    fire(0, rows_a, next_a, sem_a, nsem_a)

    def pair_body(t, _):
        c0 = t * 2
        fire(c0 + 1, rows_b, next_b, sem_b, nsem_b)
        wait(c0, rows_a, next_a, sem_a, nsem_a)
        compute(c0, rows_a, next_a)

        @pl.when(t < NCHUNK // 2 - 1)
        def _():
            fire(c0 + 2, rows_a, next_a, sem_a, nsem_a)

        wait(c0 + 1, rows_b, next_b, sem_b, nsem_b)
        compute(c0 + 1, rows_b, next_b)
        return 0

    lax.fori_loop(0, NCHUNK // 2, pair_body, 0)
    pltpu.sync_copy(out_v, out_hbm.at[pl.ds(wid * ROWS_W, ROWS_W)])


def kernel(item_seq, next_item, item_embed):
    mesh = plsc.VectorSubcoreMesh(core_axis_name="c", subcore_axis_name="s",
                                  num_cores=NC, num_subcores=NS)

    relayout = pl.kernel(
        _relayout_kernel,
        out_type=jax.ShapeDtypeStruct((PACKED,), jnp.int32),
        mesh=mesh,
        compiler_params=pltpu.CompilerParams(needs_layout_passes=False,
                                             use_tc_tiling_on_sc=True),
        scratch_types=[
            pltpu.VMEM((4, 8, BLK), jnp.float32),
            pltpu.VMEM((4, 8, BLK), jnp.float32),
            pltpu.VMEM((BLK * WROW,), jnp.int32),
            pltpu.VMEM((BLK * WROW,), jnp.int32),
            pltpu.SemaphoreType.DMA,
            pltpu.SemaphoreType.DMA,
            pltpu.SemaphoreType.DMA,
            pltpu.SemaphoreType.DMA,
        ],
    )
    lv = jnp.pad(item_embed[REM_START:],
                 ((0, REM_PAD - REM), (0, 0))).astype(jnp.bfloat16)
    lo = lax.bitcast_convert_type(lv[:, :L], jnp.uint16).astype(jnp.uint32)
    hi = lax.bitcast_convert_type(lv[:, L:], jnp.uint16).astype(jnp.uint32)
    leftover = (lo | (hi << 16)).astype(jnp.int32).reshape(REM_PAD * WROW)
    packed = relayout(jnp.transpose(item_embed), leftover)

    score = pl.kernel(
        _sbpr_kernel,
        out_type=jax.ShapeDtypeStruct((BATCH,), jnp.float32),
        mesh=mesh,
        compiler_params=pltpu.CompilerParams(needs_layout_passes=False,
                                             use_tc_tiling_on_sc=False),
        scratch_types=[
            pltpu.VMEM((IDX_W,), jnp.int32),
            pltpu.VMEM((ROWS_W,), jnp.int32),
            pltpu.VMEM((IDX_C, WROW), jnp.int32),
            pltpu.VMEM((IDX_C, WROW), jnp.int32),
            pltpu.VMEM((CHUNK, WROW), jnp.int32),
            pltpu.VMEM((CHUNK, WROW), jnp.int32),
            pltpu.VMEM((CHUNK * EMBED_DIM,), jnp.float32),
            pltpu.VMEM((ROWS_W,), jnp.float32),
            pltpu.SemaphoreType.DMA,
            pltpu.SemaphoreType.DMA,
            pltpu.SemaphoreType.DMA,
            pltpu.SemaphoreType.DMA,
        ],
    )
    seq_flat = item_seq.reshape(BATCH * MAX_LEN)
    table_rows = packed.reshape(ITEMS_PAD, WROW)
    return score(seq_flat, next_item, table_rows)
